# Initial kernel scaffold; baseline (speedup 1.0000x reference)
import jax
import jax.numpy as jnp
from jax.experimental import pallas as pl


def kernel(epoch, f_s, f_t, idx, contrast_idx, W_s, b_s, W_t, b_t, memory_s, memory_t):
    raise NotImplementedError("write your pallas kernel here")


# speedup vs baseline: 1.3237x; 1.3237x over previous
import functools

import jax
import jax.numpy as jnp
from jax import lax
from jax.experimental import pallas as pl
from jax.experimental.pallas import tpu as pltpu
from jax.experimental.pallas import tpu_sc as plsc

N_DATA = 100000
FEAT = 128
BSZ = 1024
K = 511
P = 1
KTOT = P + K
T_NCE = 0.07
EPS = 1e-07

NC = 2
NS = 16
NW = NC * NS
B_PER_W = BSZ // NW
KC = 128
NKC = KTOT // KC
NG = KC // 16
NFC = FEAT // 16


_BB = 256


def _emb_body(fs_ref, ft_ref, ws_ref, bs_ref, wt_ref, bt_ref, out_ref):
    es = jnp.dot(fs_ref[...], ws_ref[...],
                 preferred_element_type=jnp.float32,
                 precision=lax.Precision.HIGHEST) + bs_ref[...]
    es = es / jnp.sqrt(jnp.sum(es * es, axis=1, keepdims=True))
    et = jnp.dot(ft_ref[...], wt_ref[...],
                 preferred_element_type=jnp.float32,
                 precision=lax.Precision.HIGHEST) + bt_ref[...]
    et = et / jnp.sqrt(jnp.sum(et * et, axis=1, keepdims=True))
    out_ref[0] = es
    out_ref[1] = et


def _emb_call(f_s, f_t, W_s, b_s, W_t, b_t):
    grid = (BSZ // _BB,)
    return pl.pallas_call(
        _emb_body,
        grid=grid,
        in_specs=[
            pl.BlockSpec((_BB, f_s.shape[1]), lambda i: (i, 0)),
            pl.BlockSpec((_BB, f_t.shape[1]), lambda i: (i, 0)),
            pl.BlockSpec((f_s.shape[1], FEAT), lambda i: (0, 0)),
            pl.BlockSpec((1, FEAT), lambda i: (0, 0)),
            pl.BlockSpec((f_t.shape[1], FEAT), lambda i: (0, 0)),
            pl.BlockSpec((1, FEAT), lambda i: (0, 0)),
        ],
        out_specs=pl.BlockSpec((2, _BB, FEAT), lambda i: (0, i, 0)),
        out_shape=jax.ShapeDtypeStruct((2, BSZ, FEAT), jnp.float32),
    )(f_s, f_t, W_s, b_s.reshape(1, FEAT), W_t, b_t.reshape(1, FEAT))


_sc_mesh = plsc.VectorSubcoreMesh(
    core_axis_name="c", subcore_axis_name="s", num_cores=NC, num_subcores=NS)


@functools.partial(
    pl.kernel,
    out_type=jax.ShapeDtypeStruct((2, BSZ, KTOT), jnp.float32),
    mesh=_sc_mesh,
    compiler_params=pltpu.CompilerParams(needs_layout_passes=False),
    scratch_types=[
        pltpu.VMEM((KC,), jnp.int32),
        pltpu.VMEM((KC, FEAT), jnp.float32),
        pltpu.VMEM((FEAT,), jnp.float32),
        pltpu.VMEM((KC,), jnp.float32),
        pltpu.SemaphoreType.DMA,
    ],
)
def _sc_dots(mem_t, mem_s, inds, emb, out, idx_v, rows_v, emb_v, dots_v, sem):
    wid = lax.axis_index("s") * NC + lax.axis_index("c")
    base = wid * B_PER_W
    iota16 = lax.iota(jnp.int32, 16)

    for bank, tbl in ((0, mem_t), (1, mem_s)):
        def body_b(i, _, bank=bank, tbl=tbl):
            b = base + i
            pltpu.sync_copy(emb.at[bank, b], emb_v)

            def body_kc(kc, _):
                pltpu.sync_copy(inds.at[b, pl.ds(kc * KC, KC)], idx_v)
                pltpu.async_copy(tbl.at[idx_v], rows_v, sem).wait()

                accs = tuple(jnp.zeros((16,), jnp.float32) for _ in range(NG))

                def body_c(c, accs):
                    e_c = emb_v[pl.ds(c * 16, 16)]
                    new = []
                    for g in range(NG):
                        acc = accs[g]
                        rows_g = g * 16 + iota16
                        for j in range(16):
                            col = jnp.full((16,), c * 16 + j, jnp.int32)
                            val = plsc.load_gather(rows_v, [rows_g, col])
                            s = lax.squeeze(lax.slice(e_c, (j,), (j + 1,)), (0,))
                            acc = acc + val * s
                        new.append(acc)
                    return tuple(new)

                accs = lax.fori_loop(0, NFC, body_c, accs)
                for g in range(NG):
                    dots_v[pl.ds(g * 16, 16)] = accs[g]
                pltpu.sync_copy(dots_v, out.at[bank, b, pl.ds(kc * KC, KC)])
                return 0

            lax.fori_loop(0, NKC, body_kc, 0)
            return 0

        lax.fori_loop(0, B_PER_W, body_b, 0)


def _loss_body(dots_ref, out_ref):
    m_pn = float(K) / float(N_DATA)
    denom_c = m_pn + EPS
    total = jnp.float32(0.0)
    for bank in range(2):
        x = jnp.exp(dots_ref[bank] * (1.0 / T_NCE))
        z = jnp.sum(x) * (float(N_DATA) / float(BSZ * KTOT))
        v = x / z
        log_pos = jnp.log(v / (v + denom_c))
        log_neg = jnp.log(m_pn / (v + denom_c))
        colid = lax.broadcasted_iota(jnp.int32, (BSZ, KTOT), 1)
        t = jnp.sum(jnp.where(colid < P, log_pos, log_neg))
        total = total + (-t / float(BSZ))
    out_ref[...] = jnp.broadcast_to(total, (1, 1))


def _loss_call(dots):
    return pl.pallas_call(
        _loss_body,
        out_shape=jax.ShapeDtypeStruct((1, 1), jnp.float32),
    )(dots)


def kernel(epoch, f_s, f_t, idx, contrast_idx, W_s, b_s, W_t, b_t, memory_s, memory_t):
    del epoch
    inds = jnp.concatenate([idx[:, None], contrast_idx], axis=1)
    emb = _emb_call(f_s, f_t, W_s, b_s, W_t, b_t)
    dots = _sc_dots(memory_t, memory_s, inds, emb)
    loss = _loss_call(dots)
    return loss[0, 0]


# speedup vs baseline: 1.5705x; 1.1864x over previous
import functools

import jax
import jax.numpy as jnp
from jax import lax
from jax.experimental import pallas as pl
from jax.experimental.pallas import tpu as pltpu
from jax.experimental.pallas import tpu_sc as plsc

N_DATA = 100000
FEAT = 128
BSZ = 1024
K = 511
P = 1
KTOT = P + K
T_NCE = 0.07
EPS = 1e-07

NC = 2
NS = 16
NW = NC * NS
B_PER_W = BSZ // NW
KC = 128
NKC = KTOT // KC
NG = KC // 16
NFC = FEAT // 16


_BB = 256


def _emb_body(fs_ref, ft_ref, ws_ref, bs_ref, wt_ref, bt_ref, out_ref):
    es = jnp.dot(fs_ref[...], ws_ref[...],
                 preferred_element_type=jnp.float32,
                 precision=lax.Precision.HIGHEST) + bs_ref[...]
    es = es / jnp.sqrt(jnp.sum(es * es, axis=1, keepdims=True))
    et = jnp.dot(ft_ref[...], wt_ref[...],
                 preferred_element_type=jnp.float32,
                 precision=lax.Precision.HIGHEST) + bt_ref[...]
    et = et / jnp.sqrt(jnp.sum(et * et, axis=1, keepdims=True))
    out_ref[0] = es
    out_ref[1] = et


def _emb_call(f_s, f_t, W_s, b_s, W_t, b_t):
    grid = (BSZ // _BB,)
    return pl.pallas_call(
        _emb_body,
        grid=grid,
        in_specs=[
            pl.BlockSpec((_BB, f_s.shape[1]), lambda i: (i, 0)),
            pl.BlockSpec((_BB, f_t.shape[1]), lambda i: (i, 0)),
            pl.BlockSpec((f_s.shape[1], FEAT), lambda i: (0, 0)),
            pl.BlockSpec((1, FEAT), lambda i: (0, 0)),
            pl.BlockSpec((f_t.shape[1], FEAT), lambda i: (0, 0)),
            pl.BlockSpec((1, FEAT), lambda i: (0, 0)),
        ],
        out_specs=pl.BlockSpec((2, _BB, FEAT), lambda i: (0, i, 0)),
        out_shape=jax.ShapeDtypeStruct((2, BSZ, FEAT), jnp.float32),
    )(f_s, f_t, W_s, b_s.reshape(1, FEAT), W_t, b_t.reshape(1, FEAT))


_sc_mesh = plsc.VectorSubcoreMesh(
    core_axis_name="c", subcore_axis_name="s", num_cores=NC, num_subcores=NS)


@functools.partial(
    pl.kernel,
    out_type=jax.ShapeDtypeStruct((2, BSZ, KTOT), jnp.float32),
    mesh=_sc_mesh,
    compiler_params=pltpu.CompilerParams(needs_layout_passes=False),
    scratch_types=[
        pltpu.VMEM((B_PER_W, KTOT), jnp.int32),
        pltpu.VMEM((2, KC, FEAT), jnp.float32),
        pltpu.VMEM((B_PER_W, FEAT), jnp.float32),
        pltpu.VMEM((B_PER_W, KTOT), jnp.float32),
        pltpu.SemaphoreType.DMA,
        pltpu.SemaphoreType.DMA,
    ],
)
def _sc_dots(mem_t, mem_s, inds, emb, out, idx_v, rows_v, emb_v, dots_v,
             sem0, sem1):
    wid = lax.axis_index("s") * NC + lax.axis_index("c")
    base = wid * B_PER_W
    iota16 = lax.iota(jnp.int32, 16)
    rows_g = [jnp.full((16,), g * 16, jnp.int32) + iota16 for g in range(NG)]
    sems = (sem0, sem1)

    pltpu.sync_copy(inds.at[pl.ds(base, B_PER_W)], idx_v)

    for bank, tbl in ((0, mem_t), (1, mem_s)):
        pltpu.sync_copy(emb.at[bank, pl.ds(base, B_PER_W)], emb_v)
        pltpu.async_copy(tbl.at[idx_v.at[0, pl.ds(0, KC)]], rows_v.at[0], sem0)

        def row_body(r, _, tbl=tbl):
            r16 = jnp.full((16,), r, jnp.int32)
            for kc in range(NKC):
                buf = kc % 2
                nb = (kc + 1) % 2
                if kc + 1 < NKC:
                    pltpu.async_copy(
                        tbl.at[idx_v.at[r, pl.ds((kc + 1) * KC, KC)]],
                        rows_v.at[nb], sems[nb])
                else:
                    @pl.when(r + 1 < B_PER_W)
                    def _prefetch_next_row():
                        pltpu.async_copy(
                            tbl.at[idx_v.at[r + 1, pl.ds(0, KC)]],
                            rows_v.at[0], sems[0])
                pltpu.make_async_copy(
                    tbl.at[idx_v.at[r, pl.ds(kc * KC, KC)]],
                    rows_v.at[buf], sems[buf]).wait()

                buf_ref = rows_v.at[buf]
                accs0 = tuple(jnp.zeros((16,), jnp.float32)
                              for _ in range(NG))

                def cbody(c, accs, buf_ref=buf_ref, r16=r16):
                    cb = c * 16
                    e_c = plsc.load_gather(emb_v, [r16, cb + iota16])
                    new = list(accs)
                    for j in range(16):
                        s = lax.squeeze(lax.slice(e_c, (j,), (j + 1,)), (0,))
                        col = jnp.full((16,), cb + j, jnp.int32)
                        for g in range(NG):
                            val = plsc.load_gather(buf_ref, [rows_g[g], col])
                            new[g] = new[g] + val * s
                    return tuple(new)

                accs = lax.fori_loop(0, NFC, cbody, accs0)
                for g in range(NG):
                    plsc.store_scatter(
                        dots_v, [r16, jnp.full((16,), kc * KC + g * 16,
                                               jnp.int32) + iota16], accs[g])
            return 0

        lax.fori_loop(0, B_PER_W, row_body, 0)
        pltpu.sync_copy(dots_v, out.at[bank, pl.ds(base, B_PER_W)])


def _loss_body(dots_ref, out_ref):
    m_pn = float(K) / float(N_DATA)
    denom_c = m_pn + EPS
    total = jnp.float32(0.0)
    for bank in range(2):
        x = jnp.exp(dots_ref[bank] * (1.0 / T_NCE))
        z = jnp.sum(x) * (float(N_DATA) / float(BSZ * KTOT))
        v = x / z
        log_pos = jnp.log(v / (v + denom_c))
        log_neg = jnp.log(m_pn / (v + denom_c))
        colid = lax.broadcasted_iota(jnp.int32, (BSZ, KTOT), 1)
        t = jnp.sum(jnp.where(colid < P, log_pos, log_neg))
        total = total + (-t / float(BSZ))
    out_ref[...] = jnp.broadcast_to(total, (1, 1))


def _loss_call(dots):
    return pl.pallas_call(
        _loss_body,
        out_shape=jax.ShapeDtypeStruct((1, 1), jnp.float32),
    )(dots)


def kernel(epoch, f_s, f_t, idx, contrast_idx, W_s, b_s, W_t, b_t, memory_s, memory_t):
    del epoch
    inds = jnp.concatenate([idx[:, None], contrast_idx], axis=1)
    emb = _emb_call(f_s, f_t, W_s, b_s, W_t, b_t)
    dots = _sc_dots(memory_t, memory_s, inds, emb)
    loss = _loss_call(dots)
    return loss[0, 0]
